# Initial kernel scaffold; baseline (speedup 1.0000x reference)
import functools

import jax
import jax.numpy as jnp
from jax import lax
from jax.experimental import pallas as pl
from jax.experimental.pallas import tpu as pltpu
from jax.experimental.pallas import tpu_sc as plsc

NC, NS, LANE = 2, 16, 128


def _round_up(a, b):
    return (a + b - 1) // b * b


def _segsum_counts(N_acc, R_pad):
    rpt = R_pad // (NC * NS)
    zrows = N_acc // NS
    nfull, rem = divmod(zrows, 128)
    mesh = plsc.VectorSubcoreMesh(core_axis_name="c", subcore_axis_name="s")

    @functools.partial(
        pl.kernel,
        out_type=(
            jax.ShapeDtypeStruct((NC, N_acc, 128), jnp.float32),
            jax.ShapeDtypeStruct((NC, N_acc, 16), jnp.float32),
        ),
        mesh=mesh,
        scratch_types=[
            pltpu.VMEM((LANE,), jnp.int32),
            pltpu.VMEM((LANE,), jnp.int32),
            pltpu.VMEM((LANE, 128), jnp.float32),
            pltpu.VMEM((LANE, 16), jnp.float32),
            pltpu.VMEM_SHARED((N_acc, 128), jnp.float32),
            pltpu.VMEM_SHARED((N_acc, 16), jnp.float32),
            pltpu.SemaphoreType.DMA,
        ],
    )
    def seg(x_hbm, src_hbm, dst_hbm, z128_hbm, z16_hbm, ones_hbm,
            acc_out, cnt_out, sidx, didx, rows, ones_v, acc_sh, cnt_sh, sem):
        c = lax.axis_index("c")
        s = lax.axis_index("s")
        wid = c * NS + s
        z0 = s * zrows
        for i in range(nfull):
            pltpu.sync_copy(z128_hbm, acc_sh.at[pl.ds(z0 + i * 128, 128)])
            pltpu.sync_copy(z16_hbm, cnt_sh.at[pl.ds(z0 + i * 128, 128)])
        if rem:
            pltpu.sync_copy(z128_hbm.at[pl.ds(0, rem)],
                            acc_sh.at[pl.ds(z0 + nfull * 128, rem)])
            pltpu.sync_copy(z16_hbm.at[pl.ds(0, rem)],
                            cnt_sh.at[pl.ds(z0 + nfull * 128, rem)])
        pltpu.sync_copy(ones_hbm, ones_v)
        plsc.subcore_barrier()

        base = wid * rpt

        def step(w, carry):
            r = base + w
            pltpu.sync_copy(src_hbm.at[r], sidx)
            pltpu.sync_copy(dst_hbm.at[r], didx)
            pltpu.async_copy(x_hbm.at[sidx], rows, sem).wait()
            pltpu.sync_copy(rows, acc_sh.at[didx], add=True)
            pltpu.sync_copy(ones_v, cnt_sh.at[didx], add=True)
            return carry

        lax.fori_loop(0, rpt, step, 0)
        plsc.subcore_barrier()

        for i in range(nfull):
            pltpu.sync_copy(acc_sh.at[pl.ds(z0 + i * 128, 128)], rows)
            pltpu.sync_copy(rows, acc_out.at[c, pl.ds(z0 + i * 128, 128)])
            pltpu.sync_copy(cnt_sh.at[pl.ds(z0 + i * 128, 128)], ones_v)
            pltpu.sync_copy(ones_v, cnt_out.at[c, pl.ds(z0 + i * 128, 128)])
        if rem:
            r0 = z0 + nfull * 128
            pltpu.sync_copy(acc_sh.at[pl.ds(r0, rem)], rows.at[pl.ds(0, rem)])
            pltpu.sync_copy(rows.at[pl.ds(0, rem)], acc_out.at[c, pl.ds(r0, rem)])
            pltpu.sync_copy(cnt_sh.at[pl.ds(r0, rem)], ones_v.at[pl.ds(0, rem)])
            pltpu.sync_copy(ones_v.at[pl.ds(0, rem)], cnt_out.at[c, pl.ds(r0, rem)])

    return seg


def _segsum_halves(N_acc, R_pad):
    rpt = R_pad // NS
    zrows = N_acc // NS
    nfull, rem = divmod(zrows, 128)
    mesh = plsc.VectorSubcoreMesh(core_axis_name="c", subcore_axis_name="s")

    @functools.partial(
        pl.kernel,
        out_type=jax.ShapeDtypeStruct((NC, N_acc, 128), jnp.float32),
        mesh=mesh,
        scratch_types=[
            pltpu.VMEM((LANE,), jnp.int32),
            pltpu.VMEM((LANE,), jnp.int32),
            pltpu.VMEM((LANE, 128), jnp.float32),
            pltpu.VMEM_SHARED((N_acc, 128), jnp.float32),
            pltpu.SemaphoreType.DMA,
        ],
    )
    def seg(feat_hbm, src_hbm, dst_hbm, z128_hbm,
            acc_out, sidx, didx, rows, acc_sh, sem):
        c = lax.axis_index("c")
        s = lax.axis_index("s")
        z0 = s * zrows
        for i in range(nfull):
            pltpu.sync_copy(z128_hbm, acc_sh.at[pl.ds(z0 + i * 128, 128)])
        if rem:
            pltpu.sync_copy(z128_hbm.at[pl.ds(0, rem)],
                            acc_sh.at[pl.ds(z0 + nfull * 128, rem)])
        plsc.subcore_barrier()

        sbase = c * R_pad + s * rpt
        dbase = s * rpt

        def step(w, carry):
            pltpu.sync_copy(src_hbm.at[sbase + w], sidx)
            pltpu.sync_copy(dst_hbm.at[dbase + w], didx)
            pltpu.async_copy(feat_hbm.at[sidx], rows, sem).wait()
            pltpu.sync_copy(rows, acc_sh.at[didx], add=True)
            return carry

        lax.fori_loop(0, rpt, step, 0)
        plsc.subcore_barrier()

        for i in range(nfull):
            pltpu.sync_copy(acc_sh.at[pl.ds(z0 + i * 128, 128)], rows)
            pltpu.sync_copy(rows, acc_out.at[c, pl.ds(z0 + i * 128, 128)])
        if rem:
            r0 = z0 + nfull * 128
            pltpu.sync_copy(acc_sh.at[pl.ds(r0, rem)], rows.at[pl.ds(0, rem)])
            pltpu.sync_copy(rows.at[pl.ds(0, rem)], acc_out.at[c, pl.ds(r0, rem)])

    return seg


def _dense1_body(p0, p1, c0, c1, xb, wpl, wpr, wnr, bp, bn,
                 h_out, sum_out, sq_out, acc_s, acc_q):
    i = pl.program_id(0)
    rcnt = 1.0 / jnp.maximum(c0[:, :1] + c1[:, :1], 1.0)
    agg = (p0[...] + p1[...]) * rcnt
    hp = (jnp.dot(agg, wpl[...], preferred_element_type=jnp.float32)
          + jnp.dot(xb[...], wpr[...], preferred_element_type=jnp.float32)
          + bp[...])
    hn = jnp.dot(xb[...], wnr[...], preferred_element_type=jnp.float32) + bn[...]
    hb = jnp.concatenate([hp, hn], axis=1)
    h_out[...] = hb

    @pl.when(i == 0)
    def _():
        acc_s[...] = jnp.zeros_like(acc_s)
        acc_q[...] = jnp.zeros_like(acc_q)

    acc_s[...] += jnp.sum(hb, axis=0, keepdims=True)
    acc_q[...] += jnp.sum(hb * hb, axis=0, keepdims=True)
    sum_out[...] = acc_s[...]
    sq_out[...] = acc_q[...]


def _bn_body(n_rows, hpre, sums, sq, gam, bet, out):
    mean = sums[...] / n_rows
    var = sq[...] / n_rows - mean * mean
    inv = gam[...] * lax.rsqrt(var + 1e-5)
    out[...] = jnp.maximum((hpre[...] - mean) * inv + bet[...], 0.0)[None]


def _dense2_body(ap, an, c0, c1, hp, hn, w2ap, w2rp, b2p, w2an, w2rn, b2n, out):
    rcnt = 1.0 / jnp.maximum(c0[:, :1] + c1[:, :1], 1.0)
    op = (jnp.dot(ap[...] * rcnt, w2ap[...], preferred_element_type=jnp.float32)
          + jnp.dot(hp[...], w2rp[...], preferred_element_type=jnp.float32)
          + b2p[...])
    on = (jnp.dot(an[...] * rcnt, w2an[...], preferred_element_type=jnp.float32)
          + jnp.dot(hn[...], w2rn[...], preferred_element_type=jnp.float32)
          + b2n[...])
    out[...] = jnp.concatenate([op, on], axis=1)


def kernel(x, edge_index, W_pos_l, W_pos_r, b_pos, W_neg_l, W_neg_r, b_neg,
           gamma, beta, W2_pos_l, W2_pos_r, b2_pos, W2_neg_l, W2_neg_r, b2_neg):
    N, D = x.shape
    E = edge_index.shape[1]
    H = W_pos_l.shape[1]
    R_pad = _round_up(pl.cdiv(E, LANE), NC * NS)
    N_acc = _round_up(N + 64, NS)

    src = edge_index[0]
    dst = edge_index[1]
    pad = R_pad * LANE - E
    ar = jnp.arange(pad, dtype=jnp.int32)
    src_p = jnp.concatenate([src, ar % N]).reshape(R_pad, LANE)
    dst_p = jnp.concatenate([dst, N + (ar % 64)]).reshape(R_pad, LANE)
    src_b = jnp.concatenate([src_p, src_p + N], axis=0)
    z128 = jnp.zeros((128, 128), jnp.float32)
    z16 = jnp.zeros((128, 16), jnp.float32)
    o16 = jnp.ones((128, 16), jnp.float32)

    acc_a, cnt_a = _segsum_counts(N_acc, R_pad)(x, src_p, dst_p, z128, z16, o16)
    p0, p1 = acc_a[0, :N], acc_a[1, :N]
    c0, c1 = cnt_a[0, :N], cnt_a[1, :N]

    NB = 10 if N % 10 == 0 else 1
    BR = N // NB
    mm = pl.BlockSpec((128, 128), lambda i: (0, 0))
    vb = pl.BlockSpec((1, 128), lambda i: (0, 0))
    rb = pl.BlockSpec((BR, 128), lambda i: (i, 0))
    cb = pl.BlockSpec((BR, 16), lambda i: (i, 0))
    h_pre, sums, sumsq = pl.pallas_call(
        _dense1_body,
        grid=(NB,),
        in_specs=[rb, rb, cb, cb, rb, mm, mm, mm, vb, vb],
        out_specs=[
            pl.BlockSpec((BR, 2 * H), lambda i: (i, 0)),
            pl.BlockSpec((1, 2 * H), lambda i: (0, 0)),
            pl.BlockSpec((1, 2 * H), lambda i: (0, 0)),
        ],
        out_shape=[
            jax.ShapeDtypeStruct((N, 2 * H), jnp.float32),
            jax.ShapeDtypeStruct((1, 2 * H), jnp.float32),
            jax.ShapeDtypeStruct((1, 2 * H), jnp.float32),
        ],
        scratch_shapes=[
            pltpu.VMEM((1, 2 * H), jnp.float32),
            pltpu.VMEM((1, 2 * H), jnp.float32),
        ],
    )(p0, p1, c0, c1, x, W_pos_l, W_pos_r, W_neg_r,
      b_pos.reshape(1, H), b_neg.reshape(1, H))

    h2 = pl.pallas_call(
        functools.partial(_bn_body, float(N)),
        grid=(2, NB),
        in_specs=[
            pl.BlockSpec((BR, H), lambda h, b: (b, h)),
            pl.BlockSpec((1, H), lambda h, b: (0, h)),
            pl.BlockSpec((1, H), lambda h, b: (0, h)),
            pl.BlockSpec((1, H), lambda h, b: (0, h)),
            pl.BlockSpec((1, H), lambda h, b: (0, h)),
        ],
        out_specs=pl.BlockSpec((1, BR, H), lambda h, b: (h, b, 0)),
        out_shape=jax.ShapeDtypeStruct((2, N, H), jnp.float32),
    )(h_pre, sums, sumsq, gamma.reshape(1, 2 * H), beta.reshape(1, 2 * H))

    feat2 = h2.reshape(2 * N, H)
    acc_b = _segsum_halves(N_acc, R_pad)(feat2, src_b, dst_p, z128)

    out = pl.pallas_call(
        _dense2_body,
        grid=(NB,),
        in_specs=[rb, rb, cb, cb, rb, rb, mm, mm, vb, mm, mm, vb],
        out_specs=pl.BlockSpec((BR, 2 * H), lambda i: (i, 0)),
        out_shape=jax.ShapeDtypeStruct((N, 2 * H), jnp.float32),
    )(acc_b[0, :N], acc_b[1, :N], c0, c1, h2[0], h2[1],
      W2_pos_l[:H], W2_pos_r, b2_pos.reshape(1, H),
      W2_neg_l[:H], W2_neg_r, b2_neg.reshape(1, H))
    return out


# speedup vs baseline: 10.6351x; 10.6351x over previous
import functools

import jax
import jax.numpy as jnp
from jax import lax
from jax.experimental import pallas as pl
from jax.experimental.pallas import tpu as pltpu
from jax.experimental.pallas import tpu_sc as plsc

NC, NS, LANE = 2, 16, 128
CHUNK = 40
NBUF = 2


def _round_up(a, b):
    return (a + b - 1) // b * b


def _sc_segsum(N_acc, R_pad, mode):
    rpt = R_pad // (NC * NS) if mode != "halves" else R_pad // NS
    assert rpt % CHUNK == 0
    nck = rpt // CHUNK
    zrows = N_acc // NS
    nfull, rem = divmod(zrows, 128)
    mesh = plsc.VectorSubcoreMesh(core_axis_name="c", subcore_axis_name="s",
                                  num_cores=NC, num_subcores=NS)
    gather = mode != "count"

    scratch = [pltpu.VMEM((CHUNK, LANE), jnp.int32)]
    if gather:
        scratch.append(pltpu.VMEM((CHUNK, LANE), jnp.int32))
    scratch += [pltpu.VMEM((LANE, 128), jnp.float32)
                for _ in range(NBUF if gather else 1)]
    scratch += [
        pltpu.VMEM_SHARED((N_acc, 128), jnp.float32),
        pltpu.SemaphoreType.DMA,
        pltpu.SemaphoreType.DMA,
    ]

    def body(*refs):
        if gather:
            (feat_hbm, src_hbm, dst_hbm, z128_hbm, acc_out,
             didx, sidx, *rows, acc_sh, sem_g, sem_s) = refs
        else:
            (ones_hbm, dst_hbm, z128_hbm, acc_out,
             didx, *rows, acc_sh, sem_g, sem_s) = refs
        c = lax.axis_index("c")
        s = lax.axis_index("s")
        z0 = s * zrows
        pltpu.sync_copy(z128_hbm, rows[0])
        for i in range(nfull):
            pltpu.sync_copy(rows[0], acc_sh.at[pl.ds(z0 + i * 128, 128)])
        if rem:
            pltpu.sync_copy(rows[0].at[pl.ds(0, rem)],
                            acc_sh.at[pl.ds(z0 + nfull * 128, rem)])
        plsc.subcore_barrier()

        if mode == "halves":
            sbase = c * R_pad + s * rpt
            dbase = s * rpt
        else:
            sbase = dbase = (c * NS + s) * rpt
        if not gather:
            pltpu.sync_copy(ones_hbm, rows[0])

        def step(it, carry):
            pltpu.sync_copy(dst_hbm.at[pl.ds(dbase + it * CHUNK, CHUNK)], didx)
            if gather:
                pltpu.sync_copy(src_hbm.at[pl.ds(sbase + it * CHUNK, CHUNK)],
                                sidx)
                gd, sd = {}, {}
                gd[0] = pltpu.async_copy(feat_hbm.at[sidx.at[0]], rows[0],
                                         sem_g)
                for w in range(CHUNK):
                    b = w % 2
                    if w + 1 < CHUNK:
                        if w >= 1:
                            sd[w - 1].wait()
                        gd[w + 1] = pltpu.async_copy(
                            feat_hbm.at[sidx.at[w + 1]], rows[1 - b], sem_g)
                    gd[w].wait()
                    sd[w] = pltpu.async_copy(rows[b], acc_sh.at[didx.at[w]],
                                             sem_s, add=True)
                sd[CHUNK - 2].wait()
                sd[CHUNK - 1].wait()
            else:
                sd = [pltpu.async_copy(rows[0], acc_sh.at[didx.at[j]],
                                       sem_s, add=True)
                      for j in range(CHUNK)]
                for d_ in sd:
                    d_.wait()
            return carry

        lax.fori_loop(0, nck, step, 0)
        plsc.subcore_barrier()

        for i in range(nfull):
            pltpu.sync_copy(acc_sh.at[pl.ds(z0 + i * 128, 128)], rows[0])
            pltpu.sync_copy(rows[0], acc_out.at[c, pl.ds(z0 + i * 128, 128)])
        if rem:
            r0 = z0 + nfull * 128
            pltpu.sync_copy(acc_sh.at[pl.ds(r0, rem)], rows[0].at[pl.ds(0, rem)])
            pltpu.sync_copy(rows[0].at[pl.ds(0, rem)],
                            acc_out.at[c, pl.ds(r0, rem)])

    return pl.kernel(
        body,
        out_type=jax.ShapeDtypeStruct((NC, N_acc, 128), jnp.float32),
        mesh=mesh,
        scratch_types=scratch,
    )


def _dense1_body(p0, p1, c0, c1, xb, wpl, wpr, wnr, bp, bn,
                 h_out, sum_out, sq_out, acc_s, acc_q):
    i = pl.program_id(0)
    rcnt = 1.0 / jnp.maximum(c0[:, :1] + c1[:, :1], 1.0)
    agg = (p0[...] + p1[...]) * rcnt
    hp = (jnp.dot(agg, wpl[...], preferred_element_type=jnp.float32)
          + jnp.dot(xb[...], wpr[...], preferred_element_type=jnp.float32)
          + bp[...])
    hn = jnp.dot(xb[...], wnr[...], preferred_element_type=jnp.float32) + bn[...]
    hb = jnp.concatenate([hp, hn], axis=1)
    h_out[...] = hb

    @pl.when(i == 0)
    def _():
        acc_s[...] = jnp.zeros_like(acc_s)
        acc_q[...] = jnp.zeros_like(acc_q)

    acc_s[...] += jnp.sum(hb, axis=0, keepdims=True)
    acc_q[...] += jnp.sum(hb * hb, axis=0, keepdims=True)
    sum_out[...] = acc_s[...]
    sq_out[...] = acc_q[...]


def _bn_body(n_rows, hpre, sums, sq, gam, bet, out):
    mean = sums[...] / n_rows
    var = sq[...] / n_rows - mean * mean
    inv = gam[...] * lax.rsqrt(var + 1e-5)
    out[...] = jnp.maximum((hpre[...] - mean) * inv + bet[...], 0.0)[None]


def _dense2_body(ap, an, c0, c1, hp, hn, w2ap, w2rp, b2p, w2an, w2rn, b2n, out):
    rcnt = 1.0 / jnp.maximum(c0[:, :1] + c1[:, :1], 1.0)
    op = (jnp.dot(ap[...] * rcnt, w2ap[...], preferred_element_type=jnp.float32)
          + jnp.dot(hp[...], w2rp[...], preferred_element_type=jnp.float32)
          + b2p[...])
    on = (jnp.dot(an[...] * rcnt, w2an[...], preferred_element_type=jnp.float32)
          + jnp.dot(hn[...], w2rn[...], preferred_element_type=jnp.float32)
          + b2n[...])
    out[...] = jnp.concatenate([op, on], axis=1)


def kernel(x, edge_index, W_pos_l, W_pos_r, b_pos, W_neg_l, W_neg_r, b_neg,
           gamma, beta, W2_pos_l, W2_pos_r, b2_pos, W2_neg_l, W2_neg_r, b2_neg):
    N, D = x.shape
    E = edge_index.shape[1]
    H = W_pos_l.shape[1]
    R_pad = _round_up(pl.cdiv(E, LANE), NC * NS * CHUNK)
    N_acc = _round_up(N + 64, NS * 8)

    src = edge_index[0]
    dst = edge_index[1]
    pad = R_pad * LANE - E
    ar = jnp.arange(pad, dtype=jnp.int32)
    src_p = jnp.concatenate([src, ar % N]).reshape(R_pad, LANE)
    dst_p = jnp.concatenate([dst, N + (ar % 64)]).reshape(R_pad, LANE)
    src_b = jnp.concatenate([src_p, src_p + N], axis=0)
    z128 = jnp.zeros((128, 128), jnp.float32)
    o128 = jnp.ones((128, 128), jnp.float32)

    acc_a = _sc_segsum(N_acc, R_pad, "partial")(x, src_p, dst_p, z128)
    cnt_a = _sc_segsum(N_acc, R_pad, "count")(o128, dst_p, z128)
    p0, p1 = acc_a[0, :N], acc_a[1, :N]
    c0, c1 = cnt_a[0, :N, :16], cnt_a[1, :N, :16]

    NB = 10 if N % 10 == 0 else 1
    BR = N // NB
    mm = pl.BlockSpec((128, 128), lambda i: (0, 0))
    vb = pl.BlockSpec((1, 128), lambda i: (0, 0))
    rb = pl.BlockSpec((BR, 128), lambda i: (i, 0))
    cb = pl.BlockSpec((BR, 16), lambda i: (i, 0))
    h_pre, sums, sumsq = pl.pallas_call(
        _dense1_body,
        grid=(NB,),
        in_specs=[rb, rb, cb, cb, rb, mm, mm, mm, vb, vb],
        out_specs=[
            pl.BlockSpec((BR, 2 * H), lambda i: (i, 0)),
            pl.BlockSpec((1, 2 * H), lambda i: (0, 0)),
            pl.BlockSpec((1, 2 * H), lambda i: (0, 0)),
        ],
        out_shape=[
            jax.ShapeDtypeStruct((N, 2 * H), jnp.float32),
            jax.ShapeDtypeStruct((1, 2 * H), jnp.float32),
            jax.ShapeDtypeStruct((1, 2 * H), jnp.float32),
        ],
        scratch_shapes=[
            pltpu.VMEM((1, 2 * H), jnp.float32),
            pltpu.VMEM((1, 2 * H), jnp.float32),
        ],
    )(p0, p1, c0, c1, x, W_pos_l, W_pos_r, W_neg_r,
      b_pos.reshape(1, H), b_neg.reshape(1, H))

    h2 = pl.pallas_call(
        functools.partial(_bn_body, float(N)),
        grid=(2, NB),
        in_specs=[
            pl.BlockSpec((BR, H), lambda h, b: (b, h)),
            pl.BlockSpec((1, H), lambda h, b: (0, h)),
            pl.BlockSpec((1, H), lambda h, b: (0, h)),
            pl.BlockSpec((1, H), lambda h, b: (0, h)),
            pl.BlockSpec((1, H), lambda h, b: (0, h)),
        ],
        out_specs=pl.BlockSpec((1, BR, H), lambda h, b: (h, b, 0)),
        out_shape=jax.ShapeDtypeStruct((2, N, H), jnp.float32),
    )(h_pre, sums, sumsq, gamma.reshape(1, 2 * H), beta.reshape(1, 2 * H))

    feat2 = h2.reshape(2 * N, H)
    acc_b = _sc_segsum(N_acc, R_pad, "halves")(feat2, src_b, dst_p, z128)

    out = pl.pallas_call(
        _dense2_body,
        grid=(NB,),
        in_specs=[rb, rb, cb, cb, rb, rb, mm, mm, vb, mm, mm, vb],
        out_specs=pl.BlockSpec((BR, 2 * H), lambda i: (i, 0)),
        out_shape=jax.ShapeDtypeStruct((N, 2 * H), jnp.float32),
    )(acc_b[0, :N], acc_b[1, :N], c0, c1, h2[0], h2[1],
      W2_pos_l[:H], W2_pos_r, b2_pos.reshape(1, H),
      W2_neg_l[:H], W2_neg_r, b2_neg.reshape(1, H))
    return out


# speedup vs baseline: 10.6745x; 1.0037x over previous
import functools

import jax
import jax.numpy as jnp
from jax import lax
from jax.experimental import pallas as pl
from jax.experimental.pallas import tpu as pltpu
from jax.experimental.pallas import tpu_sc as plsc

NC, NS, LANE = 2, 16, 128
CHUNK = 40
NBUF = 2


def _round_up(a, b):
    return (a + b - 1) // b * b


def _sc_segsum(N_acc, R_pad, mode):
    rpt = R_pad // (NC * NS) if mode != "halves" else R_pad // NS
    assert rpt % CHUNK == 0
    nck = rpt // CHUNK
    zrows = N_acc // NS
    nfull, rem = divmod(zrows, 128)
    mesh = plsc.VectorSubcoreMesh(core_axis_name="c", subcore_axis_name="s",
                                  num_cores=NC, num_subcores=NS)
    gather = mode != "count"

    scratch = [pltpu.VMEM((CHUNK, LANE), jnp.int32)]
    if gather:
        scratch.append(pltpu.VMEM((CHUNK, LANE), jnp.int32))
    scratch += [pltpu.VMEM((LANE, 128), jnp.float32)
                for _ in range(NBUF if gather else 1)]
    scratch += [
        pltpu.VMEM_SHARED((N_acc, 128), jnp.float32),
        pltpu.SemaphoreType.DMA,
        pltpu.SemaphoreType.DMA,
    ]

    def body(*refs):
        if gather:
            (feat_hbm, src_hbm, dst_hbm, z128_hbm, acc_out,
             didx, sidx, *rows, acc_sh, sem_g, sem_s) = refs
        else:
            (ones_hbm, dst_hbm, z128_hbm, acc_out,
             didx, *rows, acc_sh, sem_g, sem_s) = refs
        c = lax.axis_index("c")
        s = lax.axis_index("s")
        z0 = s * zrows
        pltpu.sync_copy(z128_hbm, rows[0])
        for i in range(nfull):
            pltpu.sync_copy(rows[0], acc_sh.at[pl.ds(z0 + i * 128, 128)])
        if rem:
            pltpu.sync_copy(rows[0].at[pl.ds(0, rem)],
                            acc_sh.at[pl.ds(z0 + nfull * 128, rem)])
        plsc.subcore_barrier()

        if mode == "halves":
            sbase = c * R_pad + s * rpt
            dbase = s * rpt
        else:
            sbase = dbase = (c * NS + s) * rpt
        if not gather:
            pltpu.sync_copy(ones_hbm, rows[0])

        def step(it, carry):
            pltpu.sync_copy(dst_hbm.at[pl.ds(dbase + it * CHUNK, CHUNK)], didx)
            if gather:
                pltpu.sync_copy(src_hbm.at[pl.ds(sbase + it * CHUNK, CHUNK)],
                                sidx)
                gd, sd = {}, {}
                gd[0] = pltpu.async_copy(feat_hbm.at[sidx.at[0]], rows[0],
                                         sem_g)
                for w in range(CHUNK):
                    b = w % 2
                    if w + 1 < CHUNK:
                        if w >= 1:
                            sd[w - 1].wait()
                        gd[w + 1] = pltpu.async_copy(
                            feat_hbm.at[sidx.at[w + 1]], rows[1 - b], sem_g)
                    gd[w].wait()
                    sd[w] = pltpu.async_copy(rows[b], acc_sh.at[didx.at[w]],
                                             sem_s, add=True)
                sd[CHUNK - 2].wait()
                sd[CHUNK - 1].wait()
            else:
                sd = [pltpu.async_copy(rows[0], acc_sh.at[didx.at[j]],
                                       sem_s, add=True)
                      for j in range(CHUNK)]
                for d_ in sd:
                    d_.wait()
            return carry

        lax.fori_loop(0, nck, step, 0)
        plsc.subcore_barrier()

        for i in range(nfull):
            pltpu.sync_copy(acc_sh.at[pl.ds(z0 + i * 128, 128)], rows[0])
            pltpu.sync_copy(rows[0], acc_out.at[c, pl.ds(z0 + i * 128, 128)])
        if rem:
            r0 = z0 + nfull * 128
            pltpu.sync_copy(acc_sh.at[pl.ds(r0, rem)], rows[0].at[pl.ds(0, rem)])
            pltpu.sync_copy(rows[0].at[pl.ds(0, rem)],
                            acc_out.at[c, pl.ds(r0, rem)])

    return pl.kernel(
        body,
        out_type=jax.ShapeDtypeStruct((NC, N_acc, 128), jnp.float32),
        mesh=mesh,
        scratch_types=scratch,
    )


def _xr_body(xb, wpr, wnr, bp, bn, xr_out):
    xr_out[...] = jnp.concatenate([
        jnp.dot(xb[...], wpr[...], preferred_element_type=jnp.float32) + bp[...],
        jnp.dot(xb[...], wnr[...], preferred_element_type=jnp.float32) + bn[...],
    ], axis=1)


def _hr_body(h0, h1, w2rp, w2rn, b2p, b2n, hr_out):
    hr_out[...] = jnp.concatenate([
        jnp.dot(h0[0], w2rp[...], preferred_element_type=jnp.float32) + b2p[...],
        jnp.dot(h1[0], w2rn[...], preferred_element_type=jnp.float32) + b2n[...],
    ], axis=1)


def _dense1_body(H, p0, p1, c0, c1, xr, wpl,
                 h_out, sum_out, sq_out, acc_s, acc_q):
    i = pl.program_id(0)
    rcnt = 1.0 / jnp.maximum(c0[:, :1] + c1[:, :1], 1.0)
    agg = (p0[...] + p1[...]) * rcnt
    hp = (jnp.dot(agg, wpl[...], preferred_element_type=jnp.float32)
          + xr[:, :H])
    hb = jnp.concatenate([hp, xr[:, H:]], axis=1)
    h_out[...] = hb

    @pl.when(i == 0)
    def _():
        acc_s[...] = jnp.zeros_like(acc_s)
        acc_q[...] = jnp.zeros_like(acc_q)

    acc_s[...] += jnp.sum(hb, axis=0, keepdims=True)
    acc_q[...] += jnp.sum(hb * hb, axis=0, keepdims=True)
    sum_out[...] = acc_s[...]
    sq_out[...] = acc_q[...]


def _bn_body(n_rows, hpre, sums, sq, gam, bet, out):
    mean = sums[...] / n_rows
    var = sq[...] / n_rows - mean * mean
    inv = gam[...] * lax.rsqrt(var + 1e-5)
    out[...] = jnp.maximum((hpre[...] - mean) * inv + bet[...], 0.0)[None]


def _dense2_body(H, ap, an, c0, c1, hr, w2ap, w2an, out):
    rcnt = 1.0 / jnp.maximum(c0[:, :1] + c1[:, :1], 1.0)
    op = (jnp.dot(ap[...] * rcnt, w2ap[...], preferred_element_type=jnp.float32)
          + hr[:, :H])
    on = (jnp.dot(an[...] * rcnt, w2an[...], preferred_element_type=jnp.float32)
          + hr[:, H:])
    out[...] = jnp.concatenate([op, on], axis=1)


def kernel(x, edge_index, W_pos_l, W_pos_r, b_pos, W_neg_l, W_neg_r, b_neg,
           gamma, beta, W2_pos_l, W2_pos_r, b2_pos, W2_neg_l, W2_neg_r, b2_neg):
    N, D = x.shape
    E = edge_index.shape[1]
    H = W_pos_l.shape[1]
    R_pad = _round_up(pl.cdiv(E, LANE), NC * NS * CHUNK)
    N_acc = _round_up(N + 64, NS * 8)

    src = edge_index[0]
    dst = edge_index[1]
    pad = R_pad * LANE - E
    ar = jnp.arange(pad, dtype=jnp.int32)
    src_p = jnp.concatenate([src, ar % N]).reshape(R_pad, LANE)
    dst_p = jnp.concatenate([dst, N + (ar % 64)]).reshape(R_pad, LANE)
    src_b = jnp.concatenate([src_p, src_p + N], axis=0)
    z128 = jnp.zeros((128, 128), jnp.float32)
    o128 = jnp.ones((128, 128), jnp.float32)

    NB = 10 if N % 10 == 0 else 1
    BR = N // NB
    mm = pl.BlockSpec((128, 128), lambda i: (0, 0))
    vb = pl.BlockSpec((1, 128), lambda i: (0, 0))
    rb = pl.BlockSpec((BR, 128), lambda i: (i, 0))
    cb = pl.BlockSpec((BR, 16), lambda i: (i, 0))
    r2b = pl.BlockSpec((BR, 2 * H), lambda i: (i, 0))

    xr = pl.pallas_call(
        _xr_body,
        grid=(NB,),
        in_specs=[rb, mm, mm, vb, vb],
        out_specs=r2b,
        out_shape=jax.ShapeDtypeStruct((N, 2 * H), jnp.float32),
    )(x, W_pos_r, W_neg_r, b_pos.reshape(1, H), b_neg.reshape(1, H))

    acc_a = _sc_segsum(N_acc, R_pad, "partial")(x, src_p, dst_p, z128)
    cnt_a = _sc_segsum(N_acc, R_pad, "count")(o128, dst_p, z128)
    p0, p1 = acc_a[0, :N], acc_a[1, :N]
    c0, c1 = cnt_a[0, :N, :16], cnt_a[1, :N, :16]

    h_pre, sums, sumsq = pl.pallas_call(
        functools.partial(_dense1_body, H),
        grid=(NB,),
        in_specs=[rb, rb, cb, cb, r2b, mm],
        out_specs=[
            r2b,
            pl.BlockSpec((1, 2 * H), lambda i: (0, 0)),
            pl.BlockSpec((1, 2 * H), lambda i: (0, 0)),
        ],
        out_shape=[
            jax.ShapeDtypeStruct((N, 2 * H), jnp.float32),
            jax.ShapeDtypeStruct((1, 2 * H), jnp.float32),
            jax.ShapeDtypeStruct((1, 2 * H), jnp.float32),
        ],
        scratch_shapes=[
            pltpu.VMEM((1, 2 * H), jnp.float32),
            pltpu.VMEM((1, 2 * H), jnp.float32),
        ],
    )(p0, p1, c0, c1, xr, W_pos_l)

    h2 = pl.pallas_call(
        functools.partial(_bn_body, float(N)),
        grid=(2, NB),
        in_specs=[
            pl.BlockSpec((BR, H), lambda h, b: (b, h)),
            pl.BlockSpec((1, H), lambda h, b: (0, h)),
            pl.BlockSpec((1, H), lambda h, b: (0, h)),
            pl.BlockSpec((1, H), lambda h, b: (0, h)),
            pl.BlockSpec((1, H), lambda h, b: (0, h)),
        ],
        out_specs=pl.BlockSpec((1, BR, H), lambda h, b: (h, b, 0)),
        out_shape=jax.ShapeDtypeStruct((2, N, H), jnp.float32),
    )(h_pre, sums, sumsq, gamma.reshape(1, 2 * H), beta.reshape(1, 2 * H))

    hr = pl.pallas_call(
        _hr_body,
        grid=(NB,),
        in_specs=[
            pl.BlockSpec((1, BR, H), lambda i: (0, i, 0)),
            pl.BlockSpec((1, BR, H), lambda i: (1, i, 0)),
            mm, mm, vb, vb,
        ],
        out_specs=r2b,
        out_shape=jax.ShapeDtypeStruct((N, 2 * H), jnp.float32),
    )(h2, h2, W2_pos_r, W2_neg_r, b2_pos.reshape(1, H), b2_neg.reshape(1, H))

    feat2 = h2.reshape(2 * N, H)
    acc_b = _sc_segsum(N_acc, R_pad, "halves")(feat2, src_b, dst_p, z128)

    out = pl.pallas_call(
        functools.partial(_dense2_body, H),
        grid=(NB,),
        in_specs=[rb, rb, cb, cb, r2b, mm, mm],
        out_specs=r2b,
        out_shape=jax.ShapeDtypeStruct((N, 2 * H), jnp.float32),
    )(acc_b[0, :N], acc_b[1, :N], c0, c1, hr, W2_pos_l[:H], W2_neg_l[:H])
    return out


# speedup vs baseline: 10.8118x; 1.0129x over previous
import functools

import jax
import jax.numpy as jnp
from jax import lax
from jax.experimental import pallas as pl
from jax.experimental.pallas import tpu as pltpu
from jax.experimental.pallas import tpu_sc as plsc

NC, NS, LANE = 2, 16, 128
CHUNK = 40
NBUF = 2


def _round_up(a, b):
    return (a + b - 1) // b * b


def _sc_segsum(N_acc, R_pad, mode):
    rpt = R_pad // (NC * NS) if mode != "halves" else R_pad // NS
    assert rpt % CHUNK == 0
    nck = rpt // CHUNK
    zrows = N_acc // NS
    nfull, rem = divmod(zrows, 128)
    mesh = plsc.VectorSubcoreMesh(core_axis_name="c", subcore_axis_name="s",
                                  num_cores=NC, num_subcores=NS)
    gather = mode != "count"

    scratch = [pltpu.VMEM((CHUNK, LANE), jnp.int32)]
    if gather:
        scratch.append(pltpu.VMEM((CHUNK, LANE), jnp.int32))
    scratch += [pltpu.VMEM((LANE, 128), jnp.float32) for _ in range(NBUF)]
    scratch += [
        pltpu.VMEM_SHARED((N_acc, 128), jnp.float32),
        pltpu.SemaphoreType.DMA,
        pltpu.SemaphoreType.DMA,
    ]

    def body(*refs):
        if gather:
            (feat_hbm, src_hbm, dst_hbm, z128_hbm, acc_out,
             didx, sidx, *rows, acc_sh, sem_g, sem_s) = refs
        else:
            (ones_hbm, dst_hbm, z128_hbm, acc_out,
             didx, *rows, acc_sh, sem_g, sem_s) = refs
        c = lax.axis_index("c")
        s = lax.axis_index("s")
        z0 = s * zrows
        pltpu.sync_copy(z128_hbm, rows[0])
        zd = [pltpu.async_copy(rows[0], acc_sh.at[pl.ds(z0 + i * 128, 128)],
                               sem_s)
              for i in range(nfull)]
        if rem:
            zd.append(pltpu.async_copy(rows[0].at[pl.ds(0, rem)],
                                       acc_sh.at[pl.ds(z0 + nfull * 128, rem)],
                                       sem_s))
        for d_ in zd:
            d_.wait()
        plsc.subcore_barrier()

        if mode == "halves":
            sbase = c * R_pad + s * rpt
            dbase = s * rpt
        else:
            sbase = dbase = (c * NS + s) * rpt
        if not gather:
            pltpu.sync_copy(ones_hbm, rows[0])

        def step(it, carry):
            pltpu.sync_copy(dst_hbm.at[pl.ds(dbase + it * CHUNK, CHUNK)], didx)
            if gather:
                pltpu.sync_copy(src_hbm.at[pl.ds(sbase + it * CHUNK, CHUNK)],
                                sidx)
                gd, sd = {}, {}
                gd[0] = pltpu.async_copy(feat_hbm.at[sidx.at[0]], rows[0],
                                         sem_g)
                for w in range(CHUNK):
                    b = w % 2
                    if w + 1 < CHUNK:
                        if w >= 1:
                            sd[w - 1].wait()
                        gd[w + 1] = pltpu.async_copy(
                            feat_hbm.at[sidx.at[w + 1]], rows[1 - b], sem_g)
                    gd[w].wait()
                    sd[w] = pltpu.async_copy(rows[b], acc_sh.at[didx.at[w]],
                                             sem_s, add=True)
                sd[CHUNK - 2].wait()
                sd[CHUNK - 1].wait()
            else:
                sd = [pltpu.async_copy(rows[0], acc_sh.at[didx.at[j]],
                                       sem_s, add=True)
                      for j in range(CHUNK)]
                for d_ in sd:
                    d_.wait()
            return carry

        lax.fori_loop(0, nck, step, 0)
        plsc.subcore_barrier()

        nw = nfull + (1 if rem else 0)
        hd = {}
        for i in range(nw):
            rlen = 128 if i < nfull else rem
            b = rows[i % NBUF].at[pl.ds(0, rlen)]
            if i >= NBUF:
                hd[i - NBUF].wait()
            pltpu.sync_copy(acc_sh.at[pl.ds(z0 + i * 128, rlen)], b)
            hd[i] = pltpu.async_copy(b, acc_out.at[c, pl.ds(z0 + i * 128, rlen)],
                                     sem_g)
        for i in range(max(0, nw - NBUF), nw):
            hd[i].wait()

    return pl.kernel(
        body,
        out_type=jax.ShapeDtypeStruct((NC, N_acc, 128), jnp.float32),
        mesh=mesh,
        scratch_types=scratch,
    )


def _xr_body(xb, wpr, wnr, bp, bn, xr_out):
    xr_out[...] = jnp.concatenate([
        jnp.dot(xb[...], wpr[...], preferred_element_type=jnp.float32) + bp[...],
        jnp.dot(xb[...], wnr[...], preferred_element_type=jnp.float32) + bn[...],
    ], axis=1)


def _hr_body(h0, h1, w2rp, w2rn, b2p, b2n, hr_out):
    hr_out[...] = jnp.concatenate([
        jnp.dot(h0[0], w2rp[...], preferred_element_type=jnp.float32) + b2p[...],
        jnp.dot(h1[0], w2rn[...], preferred_element_type=jnp.float32) + b2n[...],
    ], axis=1)


def _dense1_body(H, p0, p1, c0, c1, xr, wpl,
                 h_out, sum_out, sq_out, acc_s, acc_q):
    i = pl.program_id(0)
    rcnt = 1.0 / jnp.maximum(c0[:, :1] + c1[:, :1], 1.0)
    agg = (p0[...] + p1[...]) * rcnt
    hp = (jnp.dot(agg, wpl[...], preferred_element_type=jnp.float32)
          + xr[:, :H])
    hb = jnp.concatenate([hp, xr[:, H:]], axis=1)
    h_out[...] = hb

    @pl.when(i == 0)
    def _():
        acc_s[...] = jnp.zeros_like(acc_s)
        acc_q[...] = jnp.zeros_like(acc_q)

    acc_s[...] += jnp.sum(hb, axis=0, keepdims=True)
    acc_q[...] += jnp.sum(hb * hb, axis=0, keepdims=True)
    sum_out[...] = acc_s[...]
    sq_out[...] = acc_q[...]


def _bn_body(n_rows, hpre, sums, sq, gam, bet, out):
    mean = sums[...] / n_rows
    var = sq[...] / n_rows - mean * mean
    inv = gam[...] * lax.rsqrt(var + 1e-5)
    out[...] = jnp.maximum((hpre[...] - mean) * inv + bet[...], 0.0)[None]


def _dense2_body(H, ap, an, c0, c1, hr, w2ap, w2an, out):
    rcnt = 1.0 / jnp.maximum(c0[:, :1] + c1[:, :1], 1.0)
    op = (jnp.dot(ap[...] * rcnt, w2ap[...], preferred_element_type=jnp.float32)
          + hr[:, :H])
    on = (jnp.dot(an[...] * rcnt, w2an[...], preferred_element_type=jnp.float32)
          + hr[:, H:])
    out[...] = jnp.concatenate([op, on], axis=1)


def kernel(x, edge_index, W_pos_l, W_pos_r, b_pos, W_neg_l, W_neg_r, b_neg,
           gamma, beta, W2_pos_l, W2_pos_r, b2_pos, W2_neg_l, W2_neg_r, b2_neg):
    N, D = x.shape
    E = edge_index.shape[1]
    H = W_pos_l.shape[1]
    R_pad = _round_up(pl.cdiv(E, LANE), NC * NS * CHUNK)
    N_acc = _round_up(N + 64, NS * 8)

    src = edge_index[0]
    dst = edge_index[1]
    pad = R_pad * LANE - E
    ar = jnp.arange(pad, dtype=jnp.int32)
    src_p = jnp.concatenate([src, ar % N]).reshape(R_pad, LANE)
    dst_p = jnp.concatenate([dst, N + (ar % 64)]).reshape(R_pad, LANE)
    src_b = jnp.concatenate([src_p, src_p + N], axis=0)
    z128 = jnp.zeros((128, 128), jnp.float32)
    o128 = jnp.ones((128, 128), jnp.float32)

    NB = 10 if N % 10 == 0 else 1
    BR = N // NB
    mm = pl.BlockSpec((128, 128), lambda i: (0, 0))
    vb = pl.BlockSpec((1, 128), lambda i: (0, 0))
    rb = pl.BlockSpec((BR, 128), lambda i: (i, 0))
    cb = pl.BlockSpec((BR, 16), lambda i: (i, 0))
    r2b = pl.BlockSpec((BR, 2 * H), lambda i: (i, 0))

    xr = pl.pallas_call(
        _xr_body,
        grid=(NB,),
        in_specs=[rb, mm, mm, vb, vb],
        out_specs=r2b,
        out_shape=jax.ShapeDtypeStruct((N, 2 * H), jnp.float32),
    )(x, W_pos_r, W_neg_r, b_pos.reshape(1, H), b_neg.reshape(1, H))

    acc_a = _sc_segsum(N_acc, R_pad, "partial")(x, src_p, dst_p, z128)
    cnt_a = _sc_segsum(N_acc, R_pad, "count")(o128, dst_p, z128)
    p0, p1 = acc_a[0, :N], acc_a[1, :N]
    c0, c1 = cnt_a[0, :N, :16], cnt_a[1, :N, :16]

    h_pre, sums, sumsq = pl.pallas_call(
        functools.partial(_dense1_body, H),
        grid=(NB,),
        in_specs=[rb, rb, cb, cb, r2b, mm],
        out_specs=[
            r2b,
            pl.BlockSpec((1, 2 * H), lambda i: (0, 0)),
            pl.BlockSpec((1, 2 * H), lambda i: (0, 0)),
        ],
        out_shape=[
            jax.ShapeDtypeStruct((N, 2 * H), jnp.float32),
            jax.ShapeDtypeStruct((1, 2 * H), jnp.float32),
            jax.ShapeDtypeStruct((1, 2 * H), jnp.float32),
        ],
        scratch_shapes=[
            pltpu.VMEM((1, 2 * H), jnp.float32),
            pltpu.VMEM((1, 2 * H), jnp.float32),
        ],
    )(p0, p1, c0, c1, xr, W_pos_l)

    h2 = pl.pallas_call(
        functools.partial(_bn_body, float(N)),
        grid=(2, NB),
        in_specs=[
            pl.BlockSpec((BR, H), lambda h, b: (b, h)),
            pl.BlockSpec((1, H), lambda h, b: (0, h)),
            pl.BlockSpec((1, H), lambda h, b: (0, h)),
            pl.BlockSpec((1, H), lambda h, b: (0, h)),
            pl.BlockSpec((1, H), lambda h, b: (0, h)),
        ],
        out_specs=pl.BlockSpec((1, BR, H), lambda h, b: (h, b, 0)),
        out_shape=jax.ShapeDtypeStruct((2, N, H), jnp.float32),
    )(h_pre, sums, sumsq, gamma.reshape(1, 2 * H), beta.reshape(1, 2 * H))

    hr = pl.pallas_call(
        _hr_body,
        grid=(NB,),
        in_specs=[
            pl.BlockSpec((1, BR, H), lambda i: (0, i, 0)),
            pl.BlockSpec((1, BR, H), lambda i: (1, i, 0)),
            mm, mm, vb, vb,
        ],
        out_specs=r2b,
        out_shape=jax.ShapeDtypeStruct((N, 2 * H), jnp.float32),
    )(h2, h2, W2_pos_r, W2_neg_r, b2_pos.reshape(1, H), b2_neg.reshape(1, H))

    feat2 = h2.reshape(2 * N, H)
    acc_b = _sc_segsum(N_acc, R_pad, "halves")(feat2, src_b, dst_p, z128)

    out = pl.pallas_call(
        functools.partial(_dense2_body, H),
        grid=(NB,),
        in_specs=[rb, rb, cb, cb, r2b, mm, mm],
        out_specs=r2b,
        out_shape=jax.ShapeDtypeStruct((N, 2 * H), jnp.float32),
    )(acc_b[0, :N], acc_b[1, :N], c0, c1, hr, W2_pos_l[:H], W2_neg_l[:H])
    return out
